# baseline (speedup 1.0000x reference)
import functools

import jax
import jax.numpy as jnp
import numpy as np
from jax.experimental import pallas as pl

_NUM_CLASSES = 21
_MIN_SIZES = [35.84, 76.8, 153.6]
_MAX_SIZES = [76.8, 153.6, 230.4]
_ASPECT_RATIOS = [[2.0], [2.0, 3.0], [2.0, 3.0]]
_STEPS = [8, 16, 32]
_VARIANCE = [0.1, 0.2]


def _prior_level(fh, fw, ih, iw, min_size, max_size, ars, step):
    ws = [min_size, float(np.sqrt(min_size * max_size))]
    hs = [min_size, float(np.sqrt(min_size * max_size))]
    for ar in ars:
        r = float(np.sqrt(ar))
        ws.append(min_size * r); hs.append(min_size / r)
        ws.append(min_size / r); hs.append(min_size * r)
    ws = np.array(ws, dtype=np.float32); hs = np.array(hs, dtype=np.float32)
    cx = (np.arange(fw, dtype=np.float32) + 0.5) * step
    cy = (np.arange(fh, dtype=np.float32) + 0.5) * step
    cxg, cyg = np.meshgrid(cx, cy)
    cxg = cxg[:, :, None]; cyg = cyg[:, :, None]
    x1 = (cxg - ws / 2.0) / iw
    y1 = (cyg - hs / 2.0) / ih
    x2 = (cxg + ws / 2.0) / iw
    y2 = (cyg + hs / 2.0) / ih
    boxes = np.stack([x1, y1, x2, y2], axis=-1).reshape(-1, 4)
    var = np.tile(
        np.array([_VARIANCE[0], _VARIANCE[0], _VARIANCE[1], _VARIANCE[1]],
                 dtype=np.float32), (boxes.shape[0], 1))
    return np.stack([boxes.reshape(-1), var.reshape(-1)], axis=0)[None]


@functools.lru_cache(maxsize=None)
def _priors_const(ih, iw, shapes):
    outs = [
        _prior_level(fh, fw, ih, iw, _MIN_SIZES[i], _MAX_SIZES[i],
                     _ASPECT_RATIOS[i], _STEPS[i])
        for i, (fh, fw) in enumerate(shapes)
    ]
    pri = np.concatenate(outs, axis=2)
    return pri.reshape(1, 2, -1, 4).astype(np.float32)


def _conv_block(xv, w_ref, bvec, H, W, Cout):
    C, HW = xv.shape
    colw = jax.lax.broadcasted_iota(jnp.int32, (C, HW), 1) % W
    x_l = jnp.where(colw == 0, jnp.float32(0), xv)
    x_r = jnp.where(colw == W - 1, jnp.float32(0), xv)
    acc = jnp.broadcast_to(bvec, (HW, Cout)).astype(jnp.float32)
    for k in range(9):
        dh, dw = k // 3 - 1, k % 3 - 1
        src = x_l if dw == 1 else (x_r if dw == -1 else xv)
        o = dh * W + dw
        if o < 0:
            slab = jnp.concatenate(
                [jnp.zeros((C, -o), xv.dtype), src[:, :HW + o]], axis=1)
        elif o > 0:
            slab = jnp.concatenate(
                [src[:, o:], jnp.zeros((C, o), xv.dtype)], axis=1)
        else:
            slab = src
        acc = acc + jax.lax.dot_general(
            slab, w_ref[k], (((0,), (0,)), ((), ())),
            preferred_element_type=jnp.float32)
    return acc


def _head_convs(xs, wts, biases, dims):
    B = xs[0].shape[0]
    xfs = [x.reshape(x.shape[0], x.shape[1], -1) for x in xs]

    def body(x0, x1, x2, w0, w1, w2, b0, b1, b2, o0, o1, o2):
        for x_ref, w_ref, b_ref, o_ref, (H, W, Cout) in zip(
                (x0, x1, x2), (w0, w1, w2), (b0, b1, b2), (o0, o1, o2), dims):
            o_ref[0] = _conv_block(x_ref[0], w_ref, b_ref[0], H, W, Cout)

    in_specs, out_specs, out_shape, args = [], [], [], []
    for xf, wt, bias, (H, W, Cout) in zip(xfs, wts, biases, dims):
        C = xf.shape[1]
        in_specs.append(pl.BlockSpec((1, C, H * W), lambda b: (b, 0, 0)))
        out_specs.append(pl.BlockSpec((1, H * W, Cout), lambda b: (b, 0, 0)))
        out_shape.append(
            jax.ShapeDtypeStruct((B, H * W, Cout), jnp.float32))
        args.append(xf)
    for wt, (H, W, Cout) in zip(wts, dims):
        in_specs.append(
            pl.BlockSpec(wt.shape, lambda b: (0, 0, 0)))
        args.append(wt)
    for bias in biases:
        in_specs.append(pl.BlockSpec(bias.shape, lambda b: (0, 0)))
        args.append(bias)

    return pl.pallas_call(
        body,
        grid=(B,),
        in_specs=in_specs,
        out_specs=out_specs,
        out_shape=out_shape,
    )(*args)


def kernel(source_features_0, source_features_1, source_features_2,
           img_tensor, loc_w0, loc_b0, conf_w0, conf_b0, loc_w1, loc_b1,
           conf_w1, conf_b1, loc_w2, loc_b2, conf_w2, conf_b2):
    feats = [source_features_0, source_features_1, source_features_2]
    loc_ws = [loc_w0, loc_w1, loc_w2]; loc_bs = [loc_b0, loc_b1, loc_b2]
    conf_ws = [conf_w0, conf_w1, conf_w2]; conf_bs = [conf_b0, conf_b1, conf_b2]
    ih, iw = img_tensor.shape[2], img_tensor.shape[3]
    B = feats[0].shape[0]

    wts, biases, dims, nlocs = [], [], [], []
    for i in range(3):
        x = feats[i]
        H, W = x.shape[2], x.shape[3]
        nlocs.append(loc_ws[i].shape[0])
        wcat = jnp.concatenate([loc_ws[i], conf_ws[i]], axis=0)
        Cout = wcat.shape[0]
        wts.append(wcat.transpose(2, 3, 1, 0).reshape(9, x.shape[1], Cout))
        biases.append(jnp.concatenate([loc_bs[i], conf_bs[i]])[None, :])
        dims.append((H, W, Cout))

    ys = _head_convs(feats, wts, biases, dims)
    locs, confs = [], []
    for y, nloc in zip(ys, nlocs):
        locs.append(y[:, :, :nloc].reshape(B, -1))
        confs.append(y[:, :, nloc:].reshape(B, -1))

    loc = jnp.concatenate(locs, axis=1).reshape(B, -1, 4)
    conf = jnp.concatenate(confs, axis=1).reshape(B, -1, _NUM_CLASSES)
    shapes = tuple((f.shape[2], f.shape[3]) for f in feats)
    pri = jnp.asarray(_priors_const(ih, iw, shapes))
    return (loc, conf, pri)


# speedup vs baseline: 1.2588x; 1.2588x over previous
import functools

import jax
import jax.numpy as jnp
import numpy as np
from jax.experimental import pallas as pl

_NUM_CLASSES = 21
_MIN_SIZES = [35.84, 76.8, 153.6]
_MAX_SIZES = [76.8, 153.6, 230.4]
_ASPECT_RATIOS = [[2.0], [2.0, 3.0], [2.0, 3.0]]
_STEPS = [8, 16, 32]
_VARIANCE = [0.1, 0.2]


def _prior_level(fh, fw, ih, iw, min_size, max_size, ars, step):
    ws = [min_size, float(np.sqrt(min_size * max_size))]
    hs = [min_size, float(np.sqrt(min_size * max_size))]
    for ar in ars:
        r = float(np.sqrt(ar))
        ws.append(min_size * r); hs.append(min_size / r)
        ws.append(min_size / r); hs.append(min_size * r)
    ws = np.array(ws, dtype=np.float32); hs = np.array(hs, dtype=np.float32)
    cx = (np.arange(fw, dtype=np.float32) + 0.5) * step
    cy = (np.arange(fh, dtype=np.float32) + 0.5) * step
    cxg, cyg = np.meshgrid(cx, cy)
    cxg = cxg[:, :, None]; cyg = cyg[:, :, None]
    x1 = (cxg - ws / 2.0) / iw
    y1 = (cyg - hs / 2.0) / ih
    x2 = (cxg + ws / 2.0) / iw
    y2 = (cyg + hs / 2.0) / ih
    boxes = np.stack([x1, y1, x2, y2], axis=-1).reshape(-1, 4)
    var = np.tile(
        np.array([_VARIANCE[0], _VARIANCE[0], _VARIANCE[1], _VARIANCE[1]],
                 dtype=np.float32), (boxes.shape[0], 1))
    return np.stack([boxes.reshape(-1), var.reshape(-1)], axis=0)[None]


@functools.lru_cache(maxsize=None)
def _priors_const(ih, iw, shapes):
    outs = [
        _prior_level(fh, fw, ih, iw, _MIN_SIZES[i], _MAX_SIZES[i],
                     _ASPECT_RATIOS[i], _STEPS[i])
        for i, (fh, fw) in enumerate(shapes)
    ]
    pri = np.concatenate(outs, axis=2)
    return pri.reshape(1, 2, -1, 4).astype(np.float32)


def _conv_block(xv, w_ref, bvec, H, W, Cout):
    C, HW = xv.shape
    colw = jax.lax.broadcasted_iota(jnp.int32, (C, HW), 1) % W
    x_l = jnp.where(colw == 0, jnp.float32(0), xv)
    x_r = jnp.where(colw == W - 1, jnp.float32(0), xv)
    acc = jnp.broadcast_to(bvec, (HW, Cout)).astype(jnp.float32)
    for k in range(9):
        dh, dw = k // 3 - 1, k % 3 - 1
        src = x_l if dw == 1 else (x_r if dw == -1 else xv)
        o = dh * W + dw
        if o < 0:
            slab = jnp.concatenate(
                [jnp.zeros((C, -o), xv.dtype), src[:, :HW + o]], axis=1)
        elif o > 0:
            slab = jnp.concatenate(
                [src[:, o:], jnp.zeros((C, o), xv.dtype)], axis=1)
        else:
            slab = src
        acc = acc + jax.lax.dot_general(
            slab, w_ref[k], (((0,), (0,)), ((), ())),
            preferred_element_type=jnp.float32)
    return acc


def _head_convs(xs, wts, biases, dims):
    B = xs[0].shape[0]
    xfs = [x.reshape(x.shape[0], x.shape[1], -1) for x in xs]

    def body(x0, x1, x2, w0, w1, w2, b0, b1, b2, o0, o1, o2):
        for x_ref, w_ref, b_ref, o_ref, (H, W, Cout) in zip(
                (x0, x1, x2), (w0, w1, w2), (b0, b1, b2), (o0, o1, o2), dims):
            o_ref[0] = _conv_block(
                x_ref[0], w_ref, b_ref[0], H, W, Cout).astype(jnp.bfloat16)

    in_specs, out_specs, out_shape, args = [], [], [], []
    for xf, wt, bias, (H, W, Cout) in zip(xfs, wts, biases, dims):
        C = xf.shape[1]
        in_specs.append(pl.BlockSpec((1, C, H * W), lambda b: (b, 0, 0)))
        out_specs.append(pl.BlockSpec((1, H * W, Cout), lambda b: (b, 0, 0)))
        out_shape.append(
            jax.ShapeDtypeStruct((B, H * W, Cout), jnp.bfloat16))
        args.append(xf)
    for wt, (H, W, Cout) in zip(wts, dims):
        in_specs.append(
            pl.BlockSpec(wt.shape, lambda b: (0, 0, 0)))
        args.append(wt)
    for bias in biases:
        in_specs.append(pl.BlockSpec(bias.shape, lambda b: (0, 0)))
        args.append(bias)

    return pl.pallas_call(
        body,
        grid=(B,),
        in_specs=in_specs,
        out_specs=out_specs,
        out_shape=out_shape,
    )(*args)


def kernel(source_features_0, source_features_1, source_features_2,
           img_tensor, loc_w0, loc_b0, conf_w0, conf_b0, loc_w1, loc_b1,
           conf_w1, conf_b1, loc_w2, loc_b2, conf_w2, conf_b2):
    feats = [source_features_0, source_features_1, source_features_2]
    loc_ws = [loc_w0, loc_w1, loc_w2]; loc_bs = [loc_b0, loc_b1, loc_b2]
    conf_ws = [conf_w0, conf_w1, conf_w2]; conf_bs = [conf_b0, conf_b1, conf_b2]
    ih, iw = img_tensor.shape[2], img_tensor.shape[3]
    B = feats[0].shape[0]

    wts, biases, dims, nlocs = [], [], [], []
    for i in range(3):
        x = feats[i]
        H, W = x.shape[2], x.shape[3]
        nlocs.append(loc_ws[i].shape[0])
        wcat = jnp.concatenate([loc_ws[i], conf_ws[i]], axis=0)
        Cout = wcat.shape[0]
        wts.append(wcat.transpose(2, 3, 1, 0).reshape(9, x.shape[1], Cout))
        biases.append(jnp.concatenate([loc_bs[i], conf_bs[i]])[None, :])
        dims.append((H, W, Cout))

    ys = _head_convs(feats, wts, biases, dims)
    locs, confs = [], []
    for y, nloc in zip(ys, nlocs):
        y = y.astype(jnp.float32)
        locs.append(y[:, :, :nloc].reshape(B, -1, 4))
        confs.append(y[:, :, nloc:].reshape(B, -1, _NUM_CLASSES))

    loc = jnp.concatenate(locs, axis=1)
    conf = jnp.concatenate(confs, axis=1)
    shapes = tuple((f.shape[2], f.shape[3]) for f in feats)
    pri = jnp.asarray(_priors_const(ih, iw, shapes))
    return (loc, conf, pri)


# speedup vs baseline: 1.2953x; 1.0290x over previous
import functools

import jax
import jax.numpy as jnp
import numpy as np
from jax.experimental import pallas as pl
from jax.experimental.pallas import tpu as pltpu

_NUM_CLASSES = 21
_MIN_SIZES = [35.84, 76.8, 153.6]
_MAX_SIZES = [76.8, 153.6, 230.4]
_ASPECT_RATIOS = [[2.0], [2.0, 3.0], [2.0, 3.0]]
_STEPS = [8, 16, 32]
_VARIANCE = [0.1, 0.2]


def _prior_level(fh, fw, ih, iw, min_size, max_size, ars, step):
    ws = [min_size, float(np.sqrt(min_size * max_size))]
    hs = [min_size, float(np.sqrt(min_size * max_size))]
    for ar in ars:
        r = float(np.sqrt(ar))
        ws.append(min_size * r); hs.append(min_size / r)
        ws.append(min_size / r); hs.append(min_size * r)
    ws = np.array(ws, dtype=np.float32); hs = np.array(hs, dtype=np.float32)
    cx = (np.arange(fw, dtype=np.float32) + 0.5) * step
    cy = (np.arange(fh, dtype=np.float32) + 0.5) * step
    cxg, cyg = np.meshgrid(cx, cy)
    cxg = cxg[:, :, None]; cyg = cyg[:, :, None]
    x1 = (cxg - ws / 2.0) / iw
    y1 = (cyg - hs / 2.0) / ih
    x2 = (cxg + ws / 2.0) / iw
    y2 = (cyg + hs / 2.0) / ih
    boxes = np.stack([x1, y1, x2, y2], axis=-1).reshape(-1, 4)
    var = np.tile(
        np.array([_VARIANCE[0], _VARIANCE[0], _VARIANCE[1], _VARIANCE[1]],
                 dtype=np.float32), (boxes.shape[0], 1))
    return np.stack([boxes.reshape(-1), var.reshape(-1)], axis=0)[None]


@functools.lru_cache(maxsize=None)
def _priors_const(ih, iw, shapes):
    outs = [
        _prior_level(fh, fw, ih, iw, _MIN_SIZES[i], _MAX_SIZES[i],
                     _ASPECT_RATIOS[i], _STEPS[i])
        for i, (fh, fw) in enumerate(shapes)
    ]
    pri = np.concatenate(outs, axis=2)
    return pri.reshape(1, 2, -1, 4).astype(np.float32)


def _conv_block(xv, w_ref, bvec, H, W, Cout):
    C, HW = xv.shape
    colw = jax.lax.broadcasted_iota(jnp.int32, (C, HW), 1) % W
    x_l = jnp.where(colw == 0, jnp.float32(0), xv)
    x_r = jnp.where(colw == W - 1, jnp.float32(0), xv)
    acc = jnp.broadcast_to(bvec, (HW, Cout)).astype(jnp.float32)
    for k in range(9):
        dh, dw = k // 3 - 1, k % 3 - 1
        src = x_l if dw == 1 else (x_r if dw == -1 else xv)
        o = dh * W + dw
        if o < 0:
            slab = jnp.concatenate(
                [jnp.zeros((C, -o), xv.dtype), src[:, :HW + o]], axis=1)
        elif o > 0:
            slab = jnp.concatenate(
                [src[:, o:], jnp.zeros((C, o), xv.dtype)], axis=1)
        else:
            slab = src
        acc = acc + jax.lax.dot_general(
            slab, w_ref[k], (((0,), (0,)), ((), ())),
            preferred_element_type=jnp.float32)
    return acc


def _head_convs(xs, wts, biases, dims):
    B = xs[0].shape[0]
    xfs = [x.reshape(x.shape[0], x.shape[1], -1) for x in xs]

    def body(x0, x1, x2, w0, w1, w2, b0, b1, b2, o0, o1, o2):
        for x_ref, w_ref, b_ref, o_ref, (H, W, Cout) in zip(
                (x0, x1, x2), (w0, w1, w2), (b0, b1, b2), (o0, o1, o2), dims):
            o_ref[0] = _conv_block(x_ref[0], w_ref, b_ref[0], H, W, Cout)

    in_specs, out_specs, out_shape, args = [], [], [], []
    for xf, wt, bias, (H, W, Cout) in zip(xfs, wts, biases, dims):
        C = xf.shape[1]
        in_specs.append(pl.BlockSpec((1, C, H * W), lambda b: (b, 0, 0)))
        out_specs.append(pl.BlockSpec((1, H * W, Cout), lambda b: (b, 0, 0)))
        out_shape.append(
            jax.ShapeDtypeStruct((B, H * W, Cout), jnp.float32))
        args.append(xf)
    for wt, (H, W, Cout) in zip(wts, dims):
        in_specs.append(
            pl.BlockSpec(wt.shape, lambda b: (0, 0, 0)))
        args.append(wt)
    for bias in biases:
        in_specs.append(pl.BlockSpec(bias.shape, lambda b: (0, 0)))
        args.append(bias)

    return pl.pallas_call(
        body,
        grid=(B,),
        in_specs=in_specs,
        out_specs=out_specs,
        out_shape=out_shape,
        compiler_params=pltpu.CompilerParams(
            dimension_semantics=("parallel",)),
    )(*args)


def kernel(source_features_0, source_features_1, source_features_2,
           img_tensor, loc_w0, loc_b0, conf_w0, conf_b0, loc_w1, loc_b1,
           conf_w1, conf_b1, loc_w2, loc_b2, conf_w2, conf_b2):
    feats = [source_features_0, source_features_1, source_features_2]
    loc_ws = [loc_w0, loc_w1, loc_w2]; loc_bs = [loc_b0, loc_b1, loc_b2]
    conf_ws = [conf_w0, conf_w1, conf_w2]; conf_bs = [conf_b0, conf_b1, conf_b2]
    ih, iw = img_tensor.shape[2], img_tensor.shape[3]
    B = feats[0].shape[0]

    wts, biases, dims, nlocs = [], [], [], []
    for i in range(3):
        x = feats[i]
        H, W = x.shape[2], x.shape[3]
        nlocs.append(loc_ws[i].shape[0])
        wcat = jnp.concatenate([loc_ws[i], conf_ws[i]], axis=0)
        Cout = wcat.shape[0]
        wts.append(wcat.transpose(2, 3, 1, 0).reshape(9, x.shape[1], Cout))
        biases.append(jnp.concatenate([loc_bs[i], conf_bs[i]])[None, :])
        dims.append((H, W, Cout))

    ys = _head_convs(feats, wts, biases, dims)
    locs, confs = [], []
    for y, nloc in zip(ys, nlocs):
        locs.append(y[:, :, :nloc].reshape(B, -1, 4))
        confs.append(y[:, :, nloc:].reshape(B, -1, _NUM_CLASSES))

    loc = jnp.concatenate(locs, axis=1)
    conf = jnp.concatenate(confs, axis=1)
    shapes = tuple((f.shape[2], f.shape[3]) for f in feats)
    pri = jnp.asarray(_priors_const(ih, iw, shapes))
    return (loc, conf, pri)


# speedup vs baseline: 1.3048x; 1.0074x over previous
import functools

import jax
import jax.numpy as jnp
import numpy as np
from jax.experimental import pallas as pl
from jax.experimental.pallas import tpu as pltpu

_NUM_CLASSES = 21
_MIN_SIZES = [35.84, 76.8, 153.6]
_MAX_SIZES = [76.8, 153.6, 230.4]
_ASPECT_RATIOS = [[2.0], [2.0, 3.0], [2.0, 3.0]]
_STEPS = [8, 16, 32]
_VARIANCE = [0.1, 0.2]


def _prior_level(fh, fw, ih, iw, min_size, max_size, ars, step):
    ws = [min_size, float(np.sqrt(min_size * max_size))]
    hs = [min_size, float(np.sqrt(min_size * max_size))]
    for ar in ars:
        r = float(np.sqrt(ar))
        ws.append(min_size * r); hs.append(min_size / r)
        ws.append(min_size / r); hs.append(min_size * r)
    ws = np.array(ws, dtype=np.float32); hs = np.array(hs, dtype=np.float32)
    cx = (np.arange(fw, dtype=np.float32) + 0.5) * step
    cy = (np.arange(fh, dtype=np.float32) + 0.5) * step
    cxg, cyg = np.meshgrid(cx, cy)
    cxg = cxg[:, :, None]; cyg = cyg[:, :, None]
    x1 = (cxg - ws / 2.0) / iw
    y1 = (cyg - hs / 2.0) / ih
    x2 = (cxg + ws / 2.0) / iw
    y2 = (cyg + hs / 2.0) / ih
    boxes = np.stack([x1, y1, x2, y2], axis=-1).reshape(-1, 4)
    var = np.tile(
        np.array([_VARIANCE[0], _VARIANCE[0], _VARIANCE[1], _VARIANCE[1]],
                 dtype=np.float32), (boxes.shape[0], 1))
    return np.stack([boxes.reshape(-1), var.reshape(-1)], axis=0)[None]


@functools.lru_cache(maxsize=None)
def _priors_const(ih, iw, shapes):
    outs = [
        _prior_level(fh, fw, ih, iw, _MIN_SIZES[i], _MAX_SIZES[i],
                     _ASPECT_RATIOS[i], _STEPS[i])
        for i, (fh, fw) in enumerate(shapes)
    ]
    pri = np.concatenate(outs, axis=2)
    return pri.reshape(1, 2, -1, 4).astype(np.float32)


def _conv_block(xv, w_ref, m_ref, bvec, H, W, Cout):
    C, HW = xv.shape
    x_l = xv * m_ref[0:1]
    x_r = xv * m_ref[1:2]
    acc = jnp.broadcast_to(bvec, (HW, Cout)).astype(jnp.float32)
    for k in range(9):
        dh, dw = k // 3 - 1, k % 3 - 1
        src = x_l if dw == 1 else (x_r if dw == -1 else xv)
        o = dh * W + dw
        if o < 0:
            slab = jnp.concatenate(
                [jnp.zeros((C, -o), xv.dtype), src[:, :HW + o]], axis=1)
        elif o > 0:
            slab = jnp.concatenate(
                [src[:, o:], jnp.zeros((C, o), xv.dtype)], axis=1)
        else:
            slab = src
        acc = acc + jax.lax.dot_general(
            slab, w_ref[k], (((0,), (0,)), ((), ())),
            preferred_element_type=jnp.float32)
    return acc


def _head_convs(xs, wts, biases, dims):
    B = xs[0].shape[0]
    xfs = [x.reshape(x.shape[0], x.shape[1], -1) for x in xs]

    def body(x0, x1, x2, w0, w1, w2, b0, b1, b2, m0, m1, m2, o0, o1, o2):
        for x_ref, w_ref, b_ref, m_ref, o_ref, (H, W, Cout) in zip(
                (x0, x1, x2), (w0, w1, w2), (b0, b1, b2), (m0, m1, m2),
                (o0, o1, o2), dims):
            o_ref[0] = _conv_block(
                x_ref[0], w_ref, m_ref, b_ref[0], H, W, Cout)

    in_specs, out_specs, out_shape, args = [], [], [], []
    for xf, wt, bias, (H, W, Cout) in zip(xfs, wts, biases, dims):
        C = xf.shape[1]
        in_specs.append(pl.BlockSpec((1, C, H * W), lambda b: (b, 0, 0)))
        out_specs.append(pl.BlockSpec((1, H * W, Cout), lambda b: (b, 0, 0)))
        out_shape.append(
            jax.ShapeDtypeStruct((B, H * W, Cout), jnp.float32))
        args.append(xf)
    for wt, (H, W, Cout) in zip(wts, dims):
        in_specs.append(
            pl.BlockSpec(wt.shape, lambda b: (0, 0, 0)))
        args.append(wt)
    for bias in biases:
        in_specs.append(pl.BlockSpec(bias.shape, lambda b: (0, 0)))
        args.append(bias)
    for (H, W, Cout) in dims:
        w_col = np.arange(H * W, dtype=np.int64) % W
        masks = np.stack([(w_col != 0).astype(np.float32),
                          (w_col != W - 1).astype(np.float32)])
        in_specs.append(pl.BlockSpec((2, H * W), lambda b: (0, 0)))
        args.append(jnp.asarray(masks))

    return pl.pallas_call(
        body,
        grid=(B,),
        in_specs=in_specs,
        out_specs=out_specs,
        out_shape=out_shape,
        compiler_params=pltpu.CompilerParams(
            dimension_semantics=("parallel",)),
    )(*args)


def kernel(source_features_0, source_features_1, source_features_2,
           img_tensor, loc_w0, loc_b0, conf_w0, conf_b0, loc_w1, loc_b1,
           conf_w1, conf_b1, loc_w2, loc_b2, conf_w2, conf_b2):
    feats = [source_features_0, source_features_1, source_features_2]
    loc_ws = [loc_w0, loc_w1, loc_w2]; loc_bs = [loc_b0, loc_b1, loc_b2]
    conf_ws = [conf_w0, conf_w1, conf_w2]; conf_bs = [conf_b0, conf_b1, conf_b2]
    ih, iw = img_tensor.shape[2], img_tensor.shape[3]
    B = feats[0].shape[0]

    wts, biases, dims, nlocs = [], [], [], []
    for i in range(3):
        x = feats[i]
        H, W = x.shape[2], x.shape[3]
        nlocs.append(loc_ws[i].shape[0])
        wcat = jnp.concatenate([loc_ws[i], conf_ws[i]], axis=0)
        Cout = wcat.shape[0]
        wts.append(wcat.transpose(2, 3, 1, 0).reshape(9, x.shape[1], Cout))
        biases.append(jnp.concatenate([loc_bs[i], conf_bs[i]])[None, :])
        dims.append((H, W, Cout))

    ys = _head_convs(feats, wts, biases, dims)
    locs, confs = [], []
    for y, nloc in zip(ys, nlocs):
        locs.append(y[:, :, :nloc].reshape(B, -1, 4))
        confs.append(y[:, :, nloc:].reshape(B, -1, _NUM_CLASSES))

    loc = jnp.concatenate(locs, axis=1)
    conf = jnp.concatenate(confs, axis=1)
    shapes = tuple((f.shape[2], f.shape[3]) for f in feats)
    pri = jnp.asarray(_priors_const(ih, iw, shapes))
    return (loc, conf, pri)


# speedup vs baseline: 1.3084x; 1.0027x over previous
import functools

import jax
import jax.numpy as jnp
import numpy as np
from jax.experimental import pallas as pl
from jax.experimental.pallas import tpu as pltpu

_NUM_CLASSES = 21
_MIN_SIZES = [35.84, 76.8, 153.6]
_MAX_SIZES = [76.8, 153.6, 230.4]
_ASPECT_RATIOS = [[2.0], [2.0, 3.0], [2.0, 3.0]]
_STEPS = [8, 16, 32]
_VARIANCE = [0.1, 0.2]


def _prior_level(fh, fw, ih, iw, min_size, max_size, ars, step):
    ws = [min_size, float(np.sqrt(min_size * max_size))]
    hs = [min_size, float(np.sqrt(min_size * max_size))]
    for ar in ars:
        r = float(np.sqrt(ar))
        ws.append(min_size * r); hs.append(min_size / r)
        ws.append(min_size / r); hs.append(min_size * r)
    ws = np.array(ws, dtype=np.float32); hs = np.array(hs, dtype=np.float32)
    cx = (np.arange(fw, dtype=np.float32) + 0.5) * step
    cy = (np.arange(fh, dtype=np.float32) + 0.5) * step
    cxg, cyg = np.meshgrid(cx, cy)
    cxg = cxg[:, :, None]; cyg = cyg[:, :, None]
    x1 = (cxg - ws / 2.0) / iw
    y1 = (cyg - hs / 2.0) / ih
    x2 = (cxg + ws / 2.0) / iw
    y2 = (cyg + hs / 2.0) / ih
    boxes = np.stack([x1, y1, x2, y2], axis=-1).reshape(-1, 4)
    var = np.tile(
        np.array([_VARIANCE[0], _VARIANCE[0], _VARIANCE[1], _VARIANCE[1]],
                 dtype=np.float32), (boxes.shape[0], 1))
    return np.stack([boxes.reshape(-1), var.reshape(-1)], axis=0)[None]


@functools.lru_cache(maxsize=None)
def _priors_const(ih, iw, shapes):
    outs = [
        _prior_level(fh, fw, ih, iw, _MIN_SIZES[i], _MAX_SIZES[i],
                     _ASPECT_RATIOS[i], _STEPS[i])
        for i, (fh, fw) in enumerate(shapes)
    ]
    pri = np.concatenate(outs, axis=2)
    return pri.reshape(1, 2, -1, 4).astype(np.float32)


def _conv_block(xv, w_ref, m_ref, bvec, H, W, Cout):
    C, HW = xv.shape
    x_l = xv * m_ref[0:1]
    x_r = xv * m_ref[1:2]
    acc = jnp.broadcast_to(bvec, (HW, Cout)).astype(jnp.float32)
    for k in range(9):
        dh, dw = k // 3 - 1, k % 3 - 1
        src = x_l if dw == 1 else (x_r if dw == -1 else xv)
        o = dh * W + dw
        if o < 0:
            slab = jnp.concatenate(
                [jnp.zeros((C, -o), xv.dtype), src[:, :HW + o]], axis=1)
        elif o > 0:
            slab = jnp.concatenate(
                [src[:, o:], jnp.zeros((C, o), xv.dtype)], axis=1)
        else:
            slab = src
        acc = acc + jax.lax.dot_general(
            slab, w_ref[k], (((0,), (0,)), ((), ())),
            preferred_element_type=jnp.float32)
    return acc


def _head_convs(xs, wts, biases, dims):
    B = xs[0].shape[0]
    xfs = [x.reshape(x.shape[0], x.shape[1], -1) for x in xs]

    def body(x0, x1, x2, w0, w1, w2, b0, b1, b2, m0, m1, m2, o0, o1, o2):
        for x_ref, w_ref, b_ref, m_ref, o_ref, (H, W, Cout) in zip(
                (x0, x1, x2), (w0, w1, w2), (b0, b1, b2), (m0, m1, m2),
                (o0, o1, o2), dims):
            o_ref[0] = _conv_block(
                x_ref[0], w_ref, m_ref, b_ref[0], H, W, Cout)

    in_specs, out_specs, out_shape, args = [], [], [], []
    for xf, wt, bias, (H, W, Cout) in zip(xfs, wts, biases, dims):
        C = xf.shape[1]
        in_specs.append(pl.BlockSpec((1, C, H * W), lambda b: (b, 0, 0)))
        out_specs.append(pl.BlockSpec((1, H * W, Cout), lambda b: (b, 0, 0)))
        out_shape.append(
            jax.ShapeDtypeStruct((B, H * W, Cout), jnp.float32))
        args.append(xf)
    for wt, (H, W, Cout) in zip(wts, dims):
        in_specs.append(
            pl.BlockSpec(wt.shape, lambda b: (0, 0, 0)))
        args.append(wt)
    for bias in biases:
        in_specs.append(pl.BlockSpec(bias.shape, lambda b: (0, 0)))
        args.append(bias)
    for (H, W, Cout) in dims:
        w_col = np.arange(H * W, dtype=np.int64) % W
        masks = np.stack([(w_col != 0).astype(np.float32),
                          (w_col != W - 1).astype(np.float32)])
        in_specs.append(pl.BlockSpec((2, H * W), lambda b: (0, 0)))
        args.append(jnp.asarray(masks))

    return pl.pallas_call(
        body,
        grid=(B,),
        in_specs=in_specs,
        out_specs=out_specs,
        out_shape=out_shape,
        compiler_params=pltpu.CompilerParams(
            dimension_semantics=("parallel",)),
    )(*args)


def kernel(source_features_0, source_features_1, source_features_2,
           img_tensor, loc_w0, loc_b0, conf_w0, conf_b0, loc_w1, loc_b1,
           conf_w1, conf_b1, loc_w2, loc_b2, conf_w2, conf_b2):
    feats = [source_features_0, source_features_1, source_features_2]
    loc_ws = [loc_w0, loc_w1, loc_w2]; loc_bs = [loc_b0, loc_b1, loc_b2]
    conf_ws = [conf_w0, conf_w1, conf_w2]; conf_bs = [conf_b0, conf_b1, conf_b2]
    ih, iw = img_tensor.shape[2], img_tensor.shape[3]
    B = feats[0].shape[0]

    wts, biases, dims, nlocs = [], [], [], []
    for i in range(3):
        x = feats[i]
        H, W = x.shape[2], x.shape[3]
        nlocs.append(loc_ws[i].shape[0])
        wcat = jnp.concatenate([conf_ws[i], loc_ws[i]], axis=0)
        Cout = wcat.shape[0]
        wts.append(wcat.transpose(2, 3, 1, 0).reshape(9, x.shape[1], Cout))
        biases.append(jnp.concatenate([conf_bs[i], loc_bs[i]])[None, :])
        dims.append((H, W, Cout))

    ys = _head_convs(feats, wts, biases, dims)
    locs, confs = [], []
    for y, nloc in zip(ys, nlocs):
        nconf = y.shape[2] - nloc
        confs.append(y[:, :, :nconf].reshape(B, -1, _NUM_CLASSES))
        locs.append(y[:, :, nconf:].reshape(B, -1, 4))

    loc = jnp.concatenate(locs, axis=1)
    conf = jnp.concatenate(confs, axis=1)
    shapes = tuple((f.shape[2], f.shape[3]) for f in feats)
    pri = jnp.asarray(_priors_const(ih, iw, shapes))
    return (loc, conf, pri)
